# baseline (speedup 1.0000x reference)
import jax
import jax.numpy as jnp
from jax.experimental import pallas as pl
from jax.experimental.pallas import tpu as pltpu

NC = 1
KT = 16


def _lstm_kernel(pt_ref, x_ref, h0_ref, ctxr_ref, wa3_ref, wacT_ref,
                 wxT_ref, w1_ref, w2_ref, bx_ref, bac_ref, bhh_ref,
                 wa_ref, wha_ref,
                 outh_ref, outctx_ref,
                 h_scr, c_scr, hist_scr, hs_scr, ctxa_scr, xp_scr):
    tb = pl.program_id(1)
    T = hist_scr.shape[0]
    A = hs_scr.shape[2]
    L = ctxa_scr.shape[0]
    Bc = h_scr.shape[0]
    D = h_scr.shape[1]
    bf16 = jnp.bfloat16

    @pl.when(tb == 0)
    def _init():
        h_scr[...] = h0_ref[0]
        c_scr[...] = jnp.zeros_like(c_scr)
        hist_scr[...] = jnp.zeros_like(hist_scr)
        hs_scr[...] = jnp.broadcast_to(
            bhh_ref[...].reshape(1, 1, A).astype(bf16), hs_scr.shape)
        ctx = ctxr_ref[0]
        ca = jnp.dot(ctx.reshape(L * Bc, ctx.shape[-1]),
                     wacT_ref[...], preferred_element_type=jnp.float32)
        ctxa_scr[...] = (ca.reshape(L, Bc, A)
                         + bac_ref[...].reshape(1, 1, A)).astype(bf16)

    xk = x_ref[0]
    xp_scr[...] = jnp.dot(xk.reshape(KT * Bc, xk.shape[-1]), wxT_ref[...],
                          preferred_element_type=jnp.float32)

    h = h_scr[...]
    c = c_scr[...]
    for k in range(KT):
        t = tb * KT + k
        hq3 = jnp.dot(h.astype(bf16), wa3_ref[...],
                      preferred_element_type=jnp.float32)
        hqa = hq3[:, :A]
        hqh = hq3[:, A:2 * A]
        hrow = hq3[:, 2 * A:]

        def _scatter(h=h, hrow=hrow, t=t):
            hist_scr[t - 1] = h.astype(bf16)
            hs_scr[t - 1] = (hrow + bhh_ref[...]).astype(bf16)
        if k == 0:
            pl.when(tb > 0)(_scatter)
        else:
            _scatter()

        sc = jnp.sum(jnp.tanh(ctxa_scr[...] + hqa.astype(bf16)[None, :, :])
                     * wa_ref[...].reshape(1, 1, A),
                     axis=-1).astype(jnp.float32)
        ec = jnp.exp(sc)
        ctx_acc = jnp.sum(ec.astype(bf16)[:, :, None] * ctxr_ref[0],
                          axis=0)
        csum = jnp.sum(ec, axis=0, keepdims=True)
        ctx_vec = ctx_acc.astype(jnp.float32) / csum.T

        sh = jnp.sum(jnp.tanh(hs_scr[...] + hqh.astype(bf16)[None, :, :])
                     * wha_ref[...].reshape(1, 1, A),
                     axis=-1).astype(jnp.float32)
        eh = jnp.exp(sh)
        hc_acc = jnp.sum(eh.astype(bf16)[:, :, None] * hist_scr[...], axis=0)
        hsum = jnp.sum(eh, axis=0, keepdims=True) + 1e-7
        h_ctx = (hc_acc.astype(jnp.float32) / hsum.T).astype(bf16)
        rows = [hist_scr[pt_ref[t, b], pl.ds(b, 1), :] for b in range(Bc)]
        par_h = jnp.concatenate(rows, axis=0)

        lhs1 = jnp.concatenate([h.astype(bf16), ctx_vec.astype(bf16)], axis=1)
        lhs2 = jnp.concatenate([par_h, h_ctx], axis=1)
        pre = (xp_scr[k * Bc:(k + 1) * Bc, :]
               + jnp.dot(lhs1, w1_ref[...], preferred_element_type=jnp.float32)
               + jnp.dot(lhs2, w2_ref[...], preferred_element_type=jnp.float32)
               + bx_ref[...])
        gi = pre[:, :D]
        gf = pre[:, D:2 * D]
        gc = pre[:, 2 * D:3 * D]
        go = pre[:, 3 * D:]
        c = jax.nn.sigmoid(gf) * c + jax.nn.sigmoid(gi) * jnp.tanh(gc)
        h = jax.nn.sigmoid(go) * jnp.tanh(c)
        outh_ref[0, k] = h
        outctx_ref[0, k] = ctx_vec.astype(jnp.float32)
    h_scr[...] = h
    c_scr[...] = c


def kernel(X, context, h0, Wx, bx, Uh, Cc, Ph, Hh, Wac, bac, Wah, wa, ba,
           Whh, bhh, Whq, wha, bha, parent_t):
    B, T, DIN = X.shape
    D = h0.shape[-1]
    L, DC = context.shape[1], context.shape[2]
    A = Wac.shape[0]
    Bc = B // NC
    f32 = jnp.float32
    bf16 = jnp.bfloat16

    WxT = Wx.T.astype(bf16)
    W1 = jnp.concatenate([Uh, Cc], axis=1).T.astype(bf16)
    W2 = jnp.concatenate([Ph, Hh], axis=1).T.astype(bf16)
    Wa3 = jnp.concatenate([Wah, Whq, Whh], axis=0).T.astype(bf16)
    WacT = Wac.T.astype(bf16)

    Xr = (X.reshape(NC, Bc, T, DIN).transpose(0, 2, 1, 3)
          .astype(bf16))
    ctxr = (context.reshape(NC, Bc, L, DC).transpose(0, 2, 1, 3)
            .astype(bf16))
    h0r = h0.reshape(NC, Bc, D)
    ptT = parent_t.astype(jnp.int32).T

    bx2 = bx.reshape(1, 4 * D).astype(f32)
    bac2 = bac.reshape(1, A).astype(f32)
    bhh2 = bhh.reshape(1, A).astype(f32)
    wa2 = wa.reshape(1, A).astype(bf16)
    wha2 = wha.reshape(1, A).astype(bf16)

    outs = pl.pallas_call(
        _lstm_kernel,
        grid_spec=pltpu.PrefetchScalarGridSpec(
            num_scalar_prefetch=1,
            grid=(NC, T // KT),
            in_specs=[
                pl.BlockSpec((1, KT, Bc, DIN), lambda i, t, pt: (i, t, 0, 0)),
                pl.BlockSpec((1, Bc, D), lambda i, t, pt: (i, 0, 0)),
                pl.BlockSpec((1, L, Bc, DC), lambda i, t, pt: (i, 0, 0, 0)),
                pl.BlockSpec((D, 3 * A), lambda i, t, pt: (0, 0)),
                pl.BlockSpec((DC, A), lambda i, t, pt: (0, 0)),
                pl.BlockSpec((DIN, 4 * D), lambda i, t, pt: (0, 0)),
                pl.BlockSpec((D + DC, 4 * D), lambda i, t, pt: (0, 0)),
                pl.BlockSpec((2 * D, 4 * D), lambda i, t, pt: (0, 0)),
                pl.BlockSpec((1, 4 * D), lambda i, t, pt: (0, 0)),
                pl.BlockSpec((1, A), lambda i, t, pt: (0, 0)),
                pl.BlockSpec((1, A), lambda i, t, pt: (0, 0)),
                pl.BlockSpec((1, A), lambda i, t, pt: (0, 0)),
                pl.BlockSpec((1, A), lambda i, t, pt: (0, 0)),
            ],
            out_specs=[
                pl.BlockSpec((1, KT, Bc, D), lambda i, t, pt: (i, t, 0, 0)),
                pl.BlockSpec((1, KT, Bc, DC), lambda i, t, pt: (i, t, 0, 0)),
            ],
            scratch_shapes=[
                pltpu.VMEM((Bc, D), f32),
                pltpu.VMEM((Bc, D), f32),
                pltpu.VMEM((T, Bc, D), bf16),
                pltpu.VMEM((T, Bc, A), bf16),
                pltpu.VMEM((L, Bc, A), bf16),
                pltpu.VMEM((KT * Bc, 4 * D), f32),
            ],
        ),
        out_shape=[
            jax.ShapeDtypeStruct((NC, T, Bc, D), f32),
            jax.ShapeDtypeStruct((NC, T, Bc, DC), f32),
        ],
        compiler_params=pltpu.CompilerParams(
            dimension_semantics=("parallel", "arbitrary"),
            vmem_limit_bytes=50 * 1024 * 1024,
        ),
        name="cond_att_lstm",
    )(ptT, Xr, h0r, ctxr, Wa3, WacT, WxT, W1, W2, bx2, bac2, bhh2, wa2,
      wha2)

    out_h = outs[0].transpose(0, 2, 1, 3).reshape(B, T, D)
    out_ctx = outs[1].transpose(0, 2, 1, 3).reshape(B, T, DC)
    return out_h, out_ctx


# speedup vs baseline: 1.1799x; 1.1799x over previous
import jax
import jax.numpy as jnp
from jax.experimental import pallas as pl
from jax.experimental.pallas import tpu as pltpu

NC = 1
KT = 4


def _lstm_kernel(pt_ref, x_ref, h0_ref, ctxr_ref, wa3_ref, wacT_ref,
                 wxT_ref, w1_ref, w2_ref, bx_ref, bac_ref, bhh_ref,
                 wa_ref, wha_ref,
                 outh_ref, outctx_ref,
                 h_scr, c_scr, hist_scr, hs_scr, ctxa_scr, xp_scr):
    tb = pl.program_id(1)
    T = hist_scr.shape[0]
    A = hs_scr.shape[2]
    L = ctxa_scr.shape[0]
    Bc = h_scr.shape[0]
    D = h_scr.shape[1]
    bf16 = jnp.bfloat16

    @pl.when(tb == 0)
    def _init():
        h_scr[...] = h0_ref[0]
        c_scr[...] = jnp.zeros_like(c_scr)
        hist_scr[...] = jnp.zeros_like(hist_scr)
        hs_scr[...] = jnp.broadcast_to(
            bhh_ref[...].reshape(1, 1, A).astype(bf16), hs_scr.shape)
        ctx = ctxr_ref[0]
        ca = jnp.dot(ctx.reshape(L * Bc, ctx.shape[-1]),
                     wacT_ref[...], preferred_element_type=jnp.float32)
        ctxa_scr[...] = (ca.reshape(L, Bc, A)
                         + bac_ref[...].reshape(1, 1, A)).astype(bf16)

    xk = x_ref[0]
    xp_scr[...] = jnp.dot(xk.reshape(KT * Bc, xk.shape[-1]), wxT_ref[...],
                          preferred_element_type=jnp.float32)

    h = h_scr[...]
    c = c_scr[...]
    for k in range(KT):
        t = tb * KT + k
        hq3 = jnp.dot(h.astype(bf16), wa3_ref[...],
                      preferred_element_type=jnp.float32)
        hqa = hq3[:, :A]
        hqh = hq3[:, A:2 * A]
        hrow = hq3[:, 2 * A:]

        def _scatter(h=h, hrow=hrow, t=t):
            hist_scr[t - 1] = h.astype(bf16)
            hs_scr[t - 1] = (hrow + bhh_ref[...]).astype(bf16)
        if k == 0:
            pl.when(tb > 0)(_scatter)
        else:
            _scatter()

        sc = jnp.sum(jnp.tanh(ctxa_scr[...] + hqa.astype(bf16)[None, :, :])
                     * wa_ref[...].reshape(1, 1, A),
                     axis=-1).astype(jnp.float32)
        ec = jnp.exp(sc)
        ctx_acc = jnp.sum(ec.astype(bf16)[:, :, None] * ctxr_ref[0],
                          axis=0)
        csum = jnp.sum(ec, axis=0, keepdims=True)
        ctx_vec = ctx_acc.astype(jnp.float32) / csum.T

        sh = jnp.sum(jnp.tanh(hs_scr[...] + hqh.astype(bf16)[None, :, :])
                     * wha_ref[...].reshape(1, 1, A),
                     axis=-1).astype(jnp.float32)
        eh = jnp.exp(sh)
        hc_acc = jnp.sum(eh.astype(bf16)[:, :, None] * hist_scr[...], axis=0)
        hsum = jnp.sum(eh, axis=0, keepdims=True) + 1e-7
        h_ctx = (hc_acc.astype(jnp.float32) / hsum.T).astype(bf16)
        rows = [hist_scr[pt_ref[t, b], pl.ds(b, 1), :] for b in range(Bc)]
        par_h = jnp.concatenate(rows, axis=0)

        lhs1 = jnp.concatenate([h.astype(bf16), ctx_vec.astype(bf16)], axis=1)
        lhs2 = jnp.concatenate([par_h, h_ctx], axis=1)
        pre = (xp_scr[k * Bc:(k + 1) * Bc, :]
               + jnp.dot(lhs1, w1_ref[...], preferred_element_type=jnp.float32)
               + jnp.dot(lhs2, w2_ref[...], preferred_element_type=jnp.float32)
               + bx_ref[...])
        gi = pre[:, :D]
        gf = pre[:, D:2 * D]
        gc = pre[:, 2 * D:3 * D]
        go = pre[:, 3 * D:]
        c = jax.nn.sigmoid(gf) * c + jax.nn.sigmoid(gi) * jnp.tanh(gc)
        h = jax.nn.sigmoid(go) * jnp.tanh(c)
        outh_ref[0, k] = h
        outctx_ref[0, k] = ctx_vec.astype(jnp.float32)
    h_scr[...] = h
    c_scr[...] = c


def kernel(X, context, h0, Wx, bx, Uh, Cc, Ph, Hh, Wac, bac, Wah, wa, ba,
           Whh, bhh, Whq, wha, bha, parent_t):
    B, T, DIN = X.shape
    D = h0.shape[-1]
    L, DC = context.shape[1], context.shape[2]
    A = Wac.shape[0]
    Bc = B // NC
    f32 = jnp.float32
    bf16 = jnp.bfloat16

    WxT = Wx.T.astype(bf16)
    W1 = jnp.concatenate([Uh, Cc], axis=1).T.astype(bf16)
    W2 = jnp.concatenate([Ph, Hh], axis=1).T.astype(bf16)
    Wa3 = jnp.concatenate([Wah, Whq, Whh], axis=0).T.astype(bf16)
    WacT = Wac.T.astype(bf16)

    Xr = (X.reshape(NC, Bc, T, DIN).transpose(0, 2, 1, 3)
          .astype(bf16))
    ctxr = (context.reshape(NC, Bc, L, DC).transpose(0, 2, 1, 3)
            .astype(bf16))
    h0r = h0.reshape(NC, Bc, D)
    ptT = parent_t.astype(jnp.int32).T

    bx2 = bx.reshape(1, 4 * D).astype(f32)
    bac2 = bac.reshape(1, A).astype(f32)
    bhh2 = bhh.reshape(1, A).astype(f32)
    wa2 = wa.reshape(1, A).astype(bf16)
    wha2 = wha.reshape(1, A).astype(bf16)

    outs = pl.pallas_call(
        _lstm_kernel,
        grid_spec=pltpu.PrefetchScalarGridSpec(
            num_scalar_prefetch=1,
            grid=(NC, T // KT),
            in_specs=[
                pl.BlockSpec((1, KT, Bc, DIN), lambda i, t, pt: (i, t, 0, 0)),
                pl.BlockSpec((1, Bc, D), lambda i, t, pt: (i, 0, 0)),
                pl.BlockSpec((1, L, Bc, DC), lambda i, t, pt: (i, 0, 0, 0)),
                pl.BlockSpec((D, 3 * A), lambda i, t, pt: (0, 0)),
                pl.BlockSpec((DC, A), lambda i, t, pt: (0, 0)),
                pl.BlockSpec((DIN, 4 * D), lambda i, t, pt: (0, 0)),
                pl.BlockSpec((D + DC, 4 * D), lambda i, t, pt: (0, 0)),
                pl.BlockSpec((2 * D, 4 * D), lambda i, t, pt: (0, 0)),
                pl.BlockSpec((1, 4 * D), lambda i, t, pt: (0, 0)),
                pl.BlockSpec((1, A), lambda i, t, pt: (0, 0)),
                pl.BlockSpec((1, A), lambda i, t, pt: (0, 0)),
                pl.BlockSpec((1, A), lambda i, t, pt: (0, 0)),
                pl.BlockSpec((1, A), lambda i, t, pt: (0, 0)),
            ],
            out_specs=[
                pl.BlockSpec((1, KT, Bc, D), lambda i, t, pt: (i, t, 0, 0)),
                pl.BlockSpec((1, KT, Bc, DC), lambda i, t, pt: (i, t, 0, 0)),
            ],
            scratch_shapes=[
                pltpu.VMEM((Bc, D), f32),
                pltpu.VMEM((Bc, D), f32),
                pltpu.VMEM((T, Bc, D), bf16),
                pltpu.VMEM((T, Bc, A), bf16),
                pltpu.VMEM((L, Bc, A), bf16),
                pltpu.VMEM((KT * Bc, 4 * D), f32),
            ],
        ),
        out_shape=[
            jax.ShapeDtypeStruct((NC, T, Bc, D), f32),
            jax.ShapeDtypeStruct((NC, T, Bc, DC), f32),
        ],
        compiler_params=pltpu.CompilerParams(
            dimension_semantics=("parallel", "arbitrary"),
            vmem_limit_bytes=50 * 1024 * 1024,
        ),
        name="cond_att_lstm",
    )(ptT, Xr, h0r, ctxr, Wa3, WacT, WxT, W1, W2, bx2, bac2, bhh2, wa2,
      wha2)

    out_h = outs[0].transpose(0, 2, 1, 3).reshape(B, T, D)
    out_ctx = outs[1].transpose(0, 2, 1, 3).reshape(B, T, DC)
    return out_h, out_ctx
